# baseline (speedup 1.0000x reference)
import functools

import jax
import jax.numpy as jnp
import numpy as np
from jax import lax
from jax.experimental import pallas as pl
from jax.experimental.pallas import tpu as pltpu
from jax.experimental.pallas import tpu_sc as plsc

BATCH = 4096
NOUT = 4096
NLANE = 16
NW = 32
ROWS_PER_W = BATCH // NW
R = 8
NBLK = ROWS_PER_W // R
GW = 1024
NGRP = NOUT // GW
CPG = GW // NLANE

_M = np.zeros((4, 16), np.float32)
for _i in (8, 9, 10, 11, 12, 13, 14, 15):
    _M[0, _i] = 1.0
for _i, _v in ((2, 1), (3, 1), (6, 1), (7, 1), (8, -1), (9, -1), (12, -1), (13, -1)):
    _M[1, _i] = _v
for _i, _v in ((4, 1), (5, 1), (6, 1), (7, 1), (8, -1), (9, -1), (10, -1), (11, -1)):
    _M[2, _i] = _v
for _i, _v in ((1, 1), (2, -1), (4, -1), (6, -2), (7, -1), (8, 1), (9, 2),
               (11, 1), (13, 1), (14, -1)):
    _M[3, _i] = _v


def _rnd_bf16_bits(v):
    u = lax.bitcast_convert_type(v, jnp.uint32)
    rounded = u + jnp.uint32(0x7FFF) + ((u >> 16) & jnp.uint32(1))
    return rounded & jnp.uint32(0xFFFF0000)


def _coef_body(m_ref, wt_ref, ia_ref, ib_ref, o_ref, oi_ref):
    wt = wt_ref[...]
    m = jnp.max(wt, axis=0, keepdims=True)
    e = jnp.exp(wt - m)
    p = e / jnp.sum(e, axis=0, keepdims=True)
    coefs = lax.dot_general(
        m_ref[...], p, (((1,), (0,)), ((), ())),
        preferred_element_type=jnp.float32)
    even = jnp.concatenate([coefs[0:1, :], coefs[2:3, :]], axis=0)
    odd = jnp.concatenate([coefs[1:2, :], coefs[3:4, :]], axis=0)
    hi = _rnd_bf16_bits(even)
    lo = _rnd_bf16_bits(odd) >> 16
    o_ref[...] = (hi | lo).astype(jnp.int32)
    oi_ref[...] = ia_ref[...] | (ib_ref[...] << 16)


_coef_call = pl.pallas_call(
    _coef_body,
    out_shape=(jax.ShapeDtypeStruct((2, NOUT), jnp.int32),
               jax.ShapeDtypeStruct((NOUT,), jnp.int32)),
)


def _sc_body(x_hbm, idx_hbm, coef_hbm, out_hbm,
             idx_v, coef_v, og0, og1, og2, og3, *xr_and_sems):
    xrows = xr_and_sems[: 2 * R]
    sx0, sx1, so0, so1, so2, so3 = xr_and_sems[2 * R:]
    mesh_nc = 2
    wid = lax.axis_index("s") * mesh_nc + lax.axis_index("c")
    base = wid * ROWS_PER_W

    xbufs = (xrows[:R], xrows[R:])
    xsems = (sx0, sx1)
    obufs = (og0, og1, og2, og3)
    osems = (so0, so1, so2, so3)

    def x_start(blk, b):
        row0 = base + blk * R
        for r in range(R):
            pltpu.async_copy(x_hbm.at[row0 + r, :], xbufs[b][r], xsems[b])

    def x_wait(blk, b):
        row0 = base + blk * R
        for r in range(R):
            pltpu.make_async_copy(
                x_hbm.at[row0 + r, :], xbufs[b][r], xsems[b]).wait()

    x_start(0, 0)
    x_start(1, 1)
    pltpu.sync_copy(idx_hbm, idx_v)
    pltpu.sync_copy(coef_hbm, coef_v)

    msk16 = jnp.full((NLANE,), 0xFFFF, jnp.int32)
    mskhi = jnp.full((NLANE,), -65536, jnp.int32)

    def blk_pair(i2, carry):
        for b in (0, 1):
            blk = i2 * 2 + b
            row0 = base + blk * R
            xb = xbufs[b]
            x_wait(blk, b)

            for g in range(NGRP):
                og = obufs[g]
                dst = out_hbm.at[pl.ds(row0, R), pl.ds(g * GW, GW)]

                @pl.when(blk >= 1)
                def _():
                    pltpu.make_async_copy(og, dst, osems[g]).wait()

                @plsc.parallel_loop(0, CPG, unroll=4)
                def chunk_body(cc):
                    s = g * GW + cc * NLANE
                    pk = idx_v[pl.ds(s, NLANE)]
                    ia = pk & msk16
                    ib = lax.shift_right_logical(pk, 16)
                    w0 = coef_v[0, pl.ds(s, NLANE)]
                    w1 = coef_v[1, pl.ds(s, NLANE)]
                    c0 = plsc.bitcast(w0 & mskhi, jnp.float32)
                    ca = plsc.bitcast(lax.shift_left(w0, 16), jnp.float32)
                    cb = plsc.bitcast(w1 & mskhi, jnp.float32)
                    cab = plsc.bitcast(lax.shift_left(w1, 16), jnp.float32)
                    avs = []
                    bvs = []
                    for r in range(R):
                        avs.append(plsc.load_gather(xb[r], [ia]))
                        bvs.append(plsc.load_gather(xb[r], [ib]))
                    for r in range(R):
                        f1 = c0 + ca * avs[r]
                        f2 = cb + cab * avs[r]
                        og[r, pl.ds(cc * NLANE, NLANE)] = f1 + bvs[r] * f2

                pltpu.async_copy(og, dst, osems[g])

            @pl.when(blk < NBLK - 2)
            def _():
                x_start(blk + 2, b)
        return carry

    lax.fori_loop(0, NBLK // 2, blk_pair, 0)

    for g in range(NGRP):
        last = out_hbm.at[pl.ds(base + (NBLK - 1) * R, R),
                          pl.ds(g * GW, GW)]
        pltpu.make_async_copy(obufs[g], last, osems[g]).wait()


_sc_call = pl.kernel(
    _sc_body,
    out_type=jax.ShapeDtypeStruct((BATCH, NOUT), jnp.float32),
    mesh=plsc.VectorSubcoreMesh(core_axis_name="c", subcore_axis_name="s"),
    compiler_params=pltpu.CompilerParams(needs_layout_passes=False),
    scratch_types=[
        pltpu.VMEM((NOUT,), jnp.int32),
        pltpu.VMEM((2, NOUT), jnp.int32),
        pltpu.VMEM((R, GW), jnp.float32),
        pltpu.VMEM((R, GW), jnp.float32),
        pltpu.VMEM((R, GW), jnp.float32),
        pltpu.VMEM((R, GW), jnp.float32),
    ] + [pltpu.VMEM((NOUT,), jnp.float32) for _ in range(2 * R)] + [
        pltpu.SemaphoreType.DMA,
        pltpu.SemaphoreType.DMA,
        pltpu.SemaphoreType.DMA,
        pltpu.SemaphoreType.DMA,
        pltpu.SemaphoreType.DMA,
        pltpu.SemaphoreType.DMA,
    ],
)


def kernel(x, weights, idx_a, idx_b):
    coef, idx_pk = _coef_call(jnp.asarray(_M), weights.T,
                              idx_a.astype(jnp.int32),
                              idx_b.astype(jnp.int32))
    return _sc_call(x, idx_pk, coef)


# speedup vs baseline: 1.5158x; 1.5158x over previous
import functools

import jax
import jax.numpy as jnp
import numpy as np
from jax import lax
from jax.experimental import pallas as pl
from jax.experimental.pallas import tpu as pltpu
from jax.experimental.pallas import tpu_sc as plsc

BATCH = 4096
NOUT = 4096
NLANE = 16
NW = 32
ROWS_PER_W = BATCH // NW
R = 8
NBLK = ROWS_PER_W // R
GW = 2048
NGRP = NOUT // GW
CPG = GW // NLANE

_M = np.zeros((4, 16), np.float32)
for _i in (8, 9, 10, 11, 12, 13, 14, 15):
    _M[0, _i] = 1.0
for _i, _v in ((2, 1), (3, 1), (6, 1), (7, 1), (8, -1), (9, -1), (12, -1), (13, -1)):
    _M[1, _i] = _v
for _i, _v in ((4, 1), (5, 1), (6, 1), (7, 1), (8, -1), (9, -1), (10, -1), (11, -1)):
    _M[2, _i] = _v
for _i, _v in ((1, 1), (2, -1), (4, -1), (6, -2), (7, -1), (8, 1), (9, 2),
               (11, 1), (13, 1), (14, -1)):
    _M[3, _i] = _v


def _rnd_bf16_bits(v):
    u = lax.bitcast_convert_type(v, jnp.uint32)
    rounded = u + jnp.uint32(0x7FFF) + ((u >> 16) & jnp.uint32(1))
    return rounded & jnp.uint32(0xFFFF0000)


def _coef_body(m_ref, wt_ref, ia_ref, ib_ref, o_ref, oi_ref):
    wt = wt_ref[...]
    m = jnp.max(wt, axis=0, keepdims=True)
    e = jnp.exp(wt - m)
    p = e / jnp.sum(e, axis=0, keepdims=True)
    coefs = lax.dot_general(
        m_ref[...], p, (((1,), (0,)), ((), ())),
        preferred_element_type=jnp.float32)
    even = jnp.concatenate([coefs[0:1, :], coefs[2:3, :]], axis=0)
    odd = jnp.concatenate([coefs[1:2, :], coefs[3:4, :]], axis=0)
    hi = _rnd_bf16_bits(even)
    lo = _rnd_bf16_bits(odd) >> 16
    o_ref[...] = (hi | lo).astype(jnp.int32)
    oi_ref[...] = ia_ref[...] | (ib_ref[...] << 16)


_coef_call = pl.pallas_call(
    _coef_body,
    out_shape=(jax.ShapeDtypeStruct((2, NOUT), jnp.int32),
               jax.ShapeDtypeStruct((NOUT,), jnp.int32)),
)


def _sc_body(x_hbm, idx_hbm, coef_hbm, out_hbm,
             idx_v, coef_v, og0, og1, *xr_and_sems):
    xrows = xr_and_sems[: 2 * R]
    sx0, sx1, so0, so1 = xr_and_sems[2 * R:]
    mesh_nc = 2
    wid = lax.axis_index("s") * mesh_nc + lax.axis_index("c")
    base = wid * ROWS_PER_W

    xbufs = (xrows[:R], xrows[R:])
    xsems = (sx0, sx1)
    obufs = (og0, og1)
    osems = (so0, so1)

    def x_start(blk, b):
        row0 = base + blk * R
        for r in range(R):
            pltpu.async_copy(x_hbm.at[row0 + r, :], xbufs[b][r], xsems[b])

    def x_wait(blk, b):
        row0 = base + blk * R
        for r in range(R):
            pltpu.make_async_copy(
                x_hbm.at[row0 + r, :], xbufs[b][r], xsems[b]).wait()

    x_start(0, 0)
    x_start(1, 1)
    pltpu.sync_copy(idx_hbm, idx_v)
    pltpu.sync_copy(coef_hbm, coef_v)

    msk16 = jnp.full((NLANE,), 0xFFFF, jnp.int32)
    mskhi = jnp.full((NLANE,), -65536, jnp.int32)

    def blk_pair(i2, carry):
        for b in (0, 1):
            blk = i2 * 2 + b
            row0 = base + blk * R
            xb = xbufs[b]
            x_wait(blk, b)

            for g in range(NGRP):
                og = obufs[g]
                dst = out_hbm.at[pl.ds(row0, R), pl.ds(g * GW, GW)]

                @pl.when(blk >= 1)
                def _():
                    pltpu.make_async_copy(og, dst, osems[g]).wait()

                @plsc.parallel_loop(0, CPG, unroll=2)
                def chunk_body(cc):
                    s = g * GW + cc * NLANE
                    pk = idx_v[pl.ds(s, NLANE)]
                    ia = pk & msk16
                    ib = lax.shift_right_logical(pk, 16)
                    w0 = coef_v[0, pl.ds(s, NLANE)]
                    w1 = coef_v[1, pl.ds(s, NLANE)]
                    c0 = plsc.bitcast(w0 & mskhi, jnp.float32)
                    ca = plsc.bitcast(lax.shift_left(w0, 16), jnp.float32)
                    cb = plsc.bitcast(w1 & mskhi, jnp.float32)
                    cab = plsc.bitcast(lax.shift_left(w1, 16), jnp.float32)
                    avs = []
                    bvs = []
                    for r in range(R):
                        avs.append(plsc.load_gather(xb[r], [ia]))
                        bvs.append(plsc.load_gather(xb[r], [ib]))
                    for r in range(R):
                        f1 = c0 + ca * avs[r]
                        f2 = cb + cab * avs[r]
                        og[r, pl.ds(cc * NLANE, NLANE)] = f1 + bvs[r] * f2

                pltpu.async_copy(og, dst, osems[g])

            @pl.when(blk < NBLK - 2)
            def _():
                x_start(blk + 2, b)
        return carry

    lax.fori_loop(0, NBLK // 2, blk_pair, 0)

    for g in range(NGRP):
        last = out_hbm.at[pl.ds(base + (NBLK - 1) * R, R),
                          pl.ds(g * GW, GW)]
        pltpu.make_async_copy(obufs[g], last, osems[g]).wait()


_sc_call = pl.kernel(
    _sc_body,
    out_type=jax.ShapeDtypeStruct((BATCH, NOUT), jnp.float32),
    mesh=plsc.VectorSubcoreMesh(core_axis_name="c", subcore_axis_name="s"),
    compiler_params=pltpu.CompilerParams(needs_layout_passes=False),
    scratch_types=[
        pltpu.VMEM((NOUT,), jnp.int32),
        pltpu.VMEM((2, NOUT), jnp.int32),
        pltpu.VMEM((R, GW), jnp.float32),
        pltpu.VMEM((R, GW), jnp.float32),
    ] + [pltpu.VMEM((NOUT,), jnp.float32) for _ in range(2 * R)] + [
        pltpu.SemaphoreType.DMA,
        pltpu.SemaphoreType.DMA,
        pltpu.SemaphoreType.DMA,
        pltpu.SemaphoreType.DMA,
    ],
)


def kernel(x, weights, idx_a, idx_b):
    coef, idx_pk = _coef_call(jnp.asarray(_M), weights.T,
                              idx_a.astype(jnp.int32),
                              idx_b.astype(jnp.int32))
    return _sc_call(x, idx_pk, coef)
